# baseline (speedup 1.0000x reference)
import functools

import jax
import jax.numpy as jnp
from jax import lax
from jax.experimental import pallas as pl
from jax.experimental.pallas import tpu as pltpu
from jax.experimental.pallas import tpu_sc as plsc

N = 10000
NPAD = 10240
F = 16
F2 = 64
NC = 2
NS = 16
NW = NC * NS
L = 16
CHUNK = 128
KCH = 80
KF = 8
KFA = 8
KC0 = 96
KC1 = 64
KD0 = 96
KD1 = 64
KCMX = max(KC0, KC1)
KDMX = max(KD0, KD1)
EP = NW * KCH * CHUNK
RPT = NPAD // NS
PADI = N + 16

_mesh = plsc.VectorSubcoreMesh(
    core_axis_name="c", subcore_axis_name="s", num_cores=NC, num_subcores=NS
)
_sc_params = pltpu.CompilerParams(use_tc_tiling_on_sc=False)


def _fill1d(ref, n, val):
    def body(i, _):
        ref[pl.ds(i * L, L)] = jnp.full((L,), val, jnp.float32)
        return 0

    lax.fori_loop(0, n // L, body, 0)


def _vrsqrt(v):
    i = jax.lax.bitcast_convert_type(v, jnp.int32)
    i = jnp.int32(0x5F3759DF) - jax.lax.shift_right_logical(i, 1)
    y = jax.lax.bitcast_convert_type(i, jnp.float32)
    for _ in range(3):
        y = y * (1.5 - 0.5 * v * y * y)
    return y


@functools.partial(
    pl.kernel,
    out_type=jax.ShapeDtypeStruct((NC, NPAD), jnp.float32),
    mesh=_mesh,
    scratch_types=[
        pltpu.VMEM((KDMX, CHUNK), jnp.int32),
        pltpu.VMEM((CHUNK,), jnp.float32),
        pltpu.VMEM((RPT,), jnp.float32),
        pltpu.VMEM_SHARED((NPAD,), jnp.float32),
        pltpu.SemaphoreType.DMA,
    ],
    compiler_params=_sc_params,
)
def _deg_kernel(col_hbm, out_hbm, colbuf, ones_v, zseg, acc_sh, sem):
    c = lax.axis_index("c")
    s = lax.axis_index("s")
    _fill1d(ones_v, CHUNK, 1.0)
    _fill1d(zseg, RPT, 0.0)
    pltpu.sync_copy(zseg, acc_sh.at[pl.ds(s * RPT, RPT)])

    @pl.when(c == 0)
    def _():
        pltpu.sync_copy(col_hbm.at[pl.ds(s * KD0, KD0), :],
                        colbuf.at[pl.ds(0, KD0), :])

    @pl.when(c != 0)
    def _():
        pltpu.sync_copy(
            col_hbm.at[pl.ds(NS * KD0 + s * KD1, KD1), :],
            colbuf.at[pl.ds(0, KD1), :],
        )

    plsc.subcore_barrier()
    nt = jnp.where(c == 0, KD0 // KF, KD1 // KF)

    def body(t, _):
        ds = []
        for i in range(KF):
            j = t * KF + i
            ds.append(pltpu.async_copy(ones_v, acc_sh.at[colbuf.at[j]], sem, add=True))
        for d in ds:
            d.wait()
        return 0

    lax.fori_loop(0, nt, body, 0)
    plsc.subcore_barrier()
    pltpu.sync_copy(acc_sh.at[pl.ds(s * RPT, RPT)], out_hbm.at[c, pl.ds(s * RPT, RPT)])


def _edge_pipeline(nb, y_sh, acc_sh, rowbuf, colbuf, msgbuf, gsem, ssem0, ssem1):
    ssems = (ssem0, ssem1)

    def issue_g(t, p):
        for i in range(KFA):
            pltpu.async_copy(y_sh.at[rowbuf.at[t * KFA + i]], msgbuf.at[p, i], gsem)

    def wait_g(p):
        for i in range(KFA):
            pltpu.make_async_copy(
                y_sh.at[rowbuf.at[i]], msgbuf.at[p, i], gsem
            ).wait()

    def issue_s(t, p):
        for i in range(KFA):
            pltpu.async_copy(
                msgbuf.at[p, i], acc_sh.at[colbuf.at[t * KFA + i]], ssems[p],
                add=True,
            )

    def wait_s(p):
        for i in range(KFA):
            pltpu.make_async_copy(
                msgbuf.at[p, i], acc_sh.at[colbuf.at[i]], ssems[p]
            ).wait()

    issue_g(0, 0)

    def pair(u, _):
        t = 2 * u
        wait_g(0)
        issue_s(t, 0)

        @pl.when(u >= 1)
        def _():
            wait_s(1)
        issue_g(t + 1, 1)
        wait_g(1)
        issue_s(t + 1, 1)

        @pl.when(t + 2 < nb)
        def _():
            wait_s(0)
            issue_g(t + 2, 0)

        return 0

    lax.fori_loop(0, nb // 2, pair, 0)
    wait_s(0)
    wait_s(1)


def _stage_and_init(c, s, seg, y_sh, acc_sh):
    sl = pl.ds(s * RPT, RPT)
    pltpu.sync_copy(seg, y_sh.at[sl, :])

    @pl.when(c == 0)
    def _():
        pltpu.sync_copy(seg, acc_sh.at[sl, :])

    @pl.when(c != 0)
    def _():
        def zb(i, _):
            seg[i, :] = jnp.zeros((F,), jnp.float32)
            return 0

        lax.fori_loop(0, RPT, zb, 0)
        pltpu.sync_copy(seg, acc_sh.at[sl, :])


@functools.partial(
    pl.kernel,
    out_type=(
        jax.ShapeDtypeStruct((NC, NPAD, F), jnp.float32),
        jax.ShapeDtypeStruct((NPAD,), jnp.float32),
        jax.ShapeDtypeStruct((NPAD, F), jnp.float32),
    ),
    mesh=_mesh,
    scratch_types=[
        pltpu.VMEM((KCMX, CHUNK), jnp.int32),
        pltpu.VMEM((KCMX, CHUNK), jnp.int32),
        pltpu.VMEM((2, KFA, CHUNK, F), jnp.float32),
        pltpu.VMEM((RPT, F), jnp.float32),
        pltpu.VMEM((RPT,), jnp.float32),
        pltpu.VMEM((RPT,), jnp.float32),
        pltpu.VMEM((RPT,), jnp.float32),
        pltpu.VMEM((RPT, F), jnp.float32),
        pltpu.VMEM_SHARED((NPAD, F), jnp.float32),
        pltpu.VMEM_SHARED((NPAD, F), jnp.float32),
        pltpu.SemaphoreType.DMA,
        pltpu.SemaphoreType.DMA,
        pltpu.SemaphoreType.DMA,
    ],
    compiler_params=_sc_params,
)
def _agg1_kernel(xw_hbm, degp_hbm, row_hbm, col_hbm, s1p_hbm, dis_hbm,
                 dexp_hbm, rowbuf, colbuf, msgbuf, seg, d0seg, d1seg, disseg,
                 dexp, acc_sh, y_sh, gsem, ssem0, ssem1):
    c = lax.axis_index("c")
    s = lax.axis_index("s")
    wid = s * NC + c
    sl = pl.ds(s * RPT, RPT)
    pltpu.sync_copy(xw_hbm.at[sl, :], seg)
    pltpu.sync_copy(degp_hbm.at[0, sl], d0seg)
    pltpu.sync_copy(degp_hbm.at[1, sl], d1seg)
    @pl.when(c == 0)
    def _():
        pltpu.sync_copy(row_hbm.at[pl.ds(s * KC0, KC0), :],
                        rowbuf.at[pl.ds(0, KC0), :])
        pltpu.sync_copy(col_hbm.at[pl.ds(s * KC0, KC0), :],
                        colbuf.at[pl.ds(0, KC0), :])

    @pl.when(c != 0)
    def _():
        base = NS * KC0 + s * KC1
        pltpu.sync_copy(row_hbm.at[pl.ds(base, KC1), :], rowbuf.at[pl.ds(0, KC1), :])
        pltpu.sync_copy(col_hbm.at[pl.ds(base, KC1), :], colbuf.at[pl.ds(0, KC1), :])

    def dbody(i, _):
        v = d0seg[pl.ds(i * L, L)] + d1seg[pl.ds(i * L, L)] + 1.0
        disseg[pl.ds(i * L, L)] = _vrsqrt(v)
        return 0

    lax.fori_loop(0, RPT // L, dbody, 0)

    def ybody(i, _):
        dv = disseg[pl.ds(i * L, L)]
        for k in range(L):
            r = i * L + k
            seg[r, :] = seg[r, :] * dv[k]
            dexp[r, :] = jax.lax.broadcast_in_dim(dv[k], (F,), ())
        return 0

    lax.fori_loop(0, RPT // L, ybody, 0)

    @pl.when(c == 0)
    def _():
        pltpu.sync_copy(disseg, dis_hbm.at[sl])
        pltpu.sync_copy(dexp, dexp_hbm.at[sl, :])

    _stage_and_init(c, s, seg, y_sh, acc_sh)
    plsc.subcore_barrier()
    nb = jnp.where(c == 0, KC0 // KFA, KC1 // KFA)
    _edge_pipeline(nb, y_sh, acc_sh, rowbuf, colbuf, msgbuf, gsem, ssem0, ssem1)
    plsc.subcore_barrier()
    pltpu.sync_copy(acc_sh.at[sl, :], s1p_hbm.at[c, sl, :])


@functools.partial(
    pl.kernel,
    out_type=jax.ShapeDtypeStruct((NC, NPAD, F), jnp.float32),
    mesh=_mesh,
    scratch_types=[
        pltpu.VMEM((KCMX, CHUNK), jnp.int32),
        pltpu.VMEM((KCMX, CHUNK), jnp.int32),
        pltpu.VMEM((2, KFA, CHUNK, F), jnp.float32),
        pltpu.VMEM((RPT, F), jnp.float32),
        pltpu.VMEM((RPT, F), jnp.float32),
        pltpu.VMEM((RPT,), jnp.float32),
        pltpu.VMEM((F,), jnp.float32),
        pltpu.VMEM_SHARED((NPAD, F), jnp.float32),
        pltpu.VMEM_SHARED((NPAD, F), jnp.float32),
        pltpu.SemaphoreType.DMA,
        pltpu.SemaphoreType.DMA,
        pltpu.SemaphoreType.DMA,
    ],
    compiler_params=_sc_params,
)
def _agg2_kernel(s1p_hbm, dis_hbm, b1_hbm, row_hbm, col_hbm, s2p_hbm,
                 rowbuf, colbuf, msgbuf, seg, p1seg, disseg, b1v,
                 acc_sh, y_sh, gsem, ssem0, ssem1):
    c = lax.axis_index("c")
    s = lax.axis_index("s")
    wid = s * NC + c
    sl = pl.ds(s * RPT, RPT)
    pltpu.sync_copy(s1p_hbm.at[0, sl, :], seg)
    pltpu.sync_copy(s1p_hbm.at[1, sl, :], p1seg)
    pltpu.sync_copy(dis_hbm.at[sl], disseg)
    pltpu.sync_copy(b1_hbm, b1v)
    @pl.when(c == 0)
    def _():
        pltpu.sync_copy(row_hbm.at[pl.ds(s * KC0, KC0), :],
                        rowbuf.at[pl.ds(0, KC0), :])
        pltpu.sync_copy(col_hbm.at[pl.ds(s * KC0, KC0), :],
                        colbuf.at[pl.ds(0, KC0), :])

    @pl.when(c != 0)
    def _():
        base = NS * KC0 + s * KC1
        pltpu.sync_copy(row_hbm.at[pl.ds(base, KC1), :], rowbuf.at[pl.ds(0, KC1), :])
        pltpu.sync_copy(col_hbm.at[pl.ds(base, KC1), :], colbuf.at[pl.ds(0, KC1), :])
    b1r = b1v[...]

    def gbody(i, _):
        dv = disseg[pl.ds(i * L, L)]
        for k in range(L):
            r = i * L + k
            d = dv[k]
            h = jnp.maximum((seg[r, :] + p1seg[r, :]) * d + b1r, 0.0)
            seg[r, :] = h * d
        return 0

    lax.fori_loop(0, RPT // L, gbody, 0)
    _stage_and_init(c, s, seg, y_sh, acc_sh)
    plsc.subcore_barrier()
    nb = jnp.where(c == 0, KC0 // KFA, KC1 // KFA)
    _edge_pipeline(nb, y_sh, acc_sh, rowbuf, colbuf, msgbuf, gsem, ssem0, ssem1)
    plsc.subcore_barrier()
    pltpu.sync_copy(acc_sh.at[sl, :], s2p_hbm.at[c, sl, :])


def _tcmm_body(xp8_ref, w1b_ref, xw8_ref):
    xw8_ref[...] = jnp.dot(
        xp8_ref[...], w1b_ref[...], preferred_element_type=jnp.float32
    )


def _tco_body(s2p8_ref, de8_ref, w2b_ref, b2b_ref, o_ref):
    t8 = (s2p8_ref[0] + s2p8_ref[1]) * de8_ref[...]
    o_ref[...] = (
        jnp.dot(t8, w2b_ref[...], preferred_element_type=jnp.float32)
        + b2b_ref[...]
    )


def _lsm_body(o_ref, out_ref):
    o = o_ref[...]
    m = jnp.max(o, axis=1, keepdims=True)
    e = o - m
    lse = jnp.log(jnp.sum(jnp.exp(e), axis=1, keepdims=True))
    out_ref[...] = e - lse


def kernel(x, edge_index, W1, b1, W2, b2):
    ei = edge_index.astype(jnp.int32)
    e = ei.shape[1]
    pad = EP - e
    col2 = jnp.concatenate([ei[1], jnp.full((pad,), PADI, jnp.int32)])
    col2 = col2.reshape(EP // CHUNK, CHUNK)
    degp = _deg_kernel(col2)

    eib = jax.lax.optimization_barrier(ei)
    row2 = jnp.concatenate([eib[0], jnp.full((pad,), PADI, jnp.int32)])
    row2 = row2.reshape(EP // CHUNK, CHUNK)
    xp = jnp.pad(x, ((0, NPAD - N), (0, 0)))
    xp8 = jnp.reshape(xp, (NPAD // 8, 8 * 128))
    eye8 = jnp.eye(8, dtype=jnp.float32)
    w1b = (eye8[:, None, :, None] * W1[None, :, None, :]).reshape(8 * 128, 8 * F)
    xw8 = pl.pallas_call(
        _tcmm_body, out_shape=jax.ShapeDtypeStruct((NPAD // 8, 8 * F), jnp.float32)
    )(xp8, w1b)
    xw = jnp.reshape(xw8, (NPAD, F))

    s1p, dis, dexp = _agg1_kernel(xw, degp, row2, col2)
    s2p = _agg2_kernel(s1p, dis, b1, row2, col2)
    s2p8 = jnp.reshape(s2p, (NC, NPAD // 8, 8 * F))
    dexp8 = jnp.reshape(dexp, (NPAD // 8, 8 * F))
    w2b = (eye8[:, None, :, None] * W2[None, :, None, :]).reshape(8 * F, 8 * F2)
    b2b = jnp.tile(b2, (8,)).reshape(1, 8 * F2)
    o512 = pl.pallas_call(
        _tco_body, out_shape=jax.ShapeDtypeStruct((NPAD // 8, 8 * F2), jnp.float32)
    )(s2p8, dexp8, w2b, b2b)
    o = jnp.reshape(o512, (NPAD, F2))
    out = pl.pallas_call(
        _lsm_body, out_shape=jax.ShapeDtypeStruct((NPAD, F2), jnp.float32)
    )(o)
    return out[:N]


# speedup vs baseline: 1.0720x; 1.0720x over previous
import functools

import jax
import jax.numpy as jnp
from jax import lax
from jax.experimental import pallas as pl
from jax.experimental.pallas import tpu as pltpu
from jax.experimental.pallas import tpu_sc as plsc

N = 10000
NPAD = 10240
F = 16
F2 = 64
NC = 2
NS = 16
NW = NC * NS
L = 16
CHUNK = 128
KCH = 80
KF = 8
KFA = 8
KC0 = 96
KC1 = 64
KD0 = 88
KD1 = 72
KCMX = max(KC0, KC1)
KDMX = max(KD0, KD1)
EP = NW * KCH * CHUNK
RPT = NPAD // NS
PADI = N + 16

_mesh = plsc.VectorSubcoreMesh(
    core_axis_name="c", subcore_axis_name="s", num_cores=NC, num_subcores=NS
)
_sc_params = pltpu.CompilerParams(use_tc_tiling_on_sc=False)


def _fill1d(ref, n, val):
    def body(i, _):
        ref[pl.ds(i * L, L)] = jnp.full((L,), val, jnp.float32)
        return 0

    lax.fori_loop(0, n // L, body, 0)


def _vrsqrt(v):
    i = jax.lax.bitcast_convert_type(v, jnp.int32)
    i = jnp.int32(0x5F3759DF) - jax.lax.shift_right_logical(i, 1)
    y = jax.lax.bitcast_convert_type(i, jnp.float32)
    for _ in range(3):
        y = y * (1.5 - 0.5 * v * y * y)
    return y


@functools.partial(
    pl.kernel,
    out_type=jax.ShapeDtypeStruct((NC, NPAD), jnp.float32),
    mesh=_mesh,
    scratch_types=[
        pltpu.VMEM((KDMX, CHUNK), jnp.int32),
        pltpu.VMEM((CHUNK,), jnp.float32),
        pltpu.VMEM((RPT,), jnp.float32),
        pltpu.VMEM_SHARED((NPAD,), jnp.float32),
        pltpu.SemaphoreType.DMA,
    ],
    compiler_params=_sc_params,
)
def _deg_kernel(col_hbm, out_hbm, colbuf, ones_v, zseg, acc_sh, sem):
    c = lax.axis_index("c")
    s = lax.axis_index("s")
    _fill1d(ones_v, CHUNK, 1.0)
    _fill1d(zseg, RPT, 0.0)
    pltpu.sync_copy(zseg, acc_sh.at[pl.ds(s * RPT, RPT)])

    @pl.when(c == 0)
    def _():
        pltpu.sync_copy(col_hbm.at[pl.ds(s * KD0, KD0), :],
                        colbuf.at[pl.ds(0, KD0), :])

    @pl.when(c != 0)
    def _():
        pltpu.sync_copy(
            col_hbm.at[pl.ds(NS * KD0 + s * KD1, KD1), :],
            colbuf.at[pl.ds(0, KD1), :],
        )

    plsc.subcore_barrier()
    nt = jnp.where(c == 0, KD0 // KF, KD1 // KF)

    def body(t, _):
        ds = []
        for i in range(KF):
            j = t * KF + i
            ds.append(pltpu.async_copy(ones_v, acc_sh.at[colbuf.at[j]], sem, add=True))
        for d in ds:
            d.wait()
        return 0

    lax.fori_loop(0, nt, body, 0)
    plsc.subcore_barrier()
    pltpu.sync_copy(acc_sh.at[pl.ds(s * RPT, RPT)], out_hbm.at[c, pl.ds(s * RPT, RPT)])


def _edge_pipeline(nb, y_sh, acc_sh, rowbuf, colbuf, msgbuf, gsem, ssem0, ssem1):
    ssems = (ssem0, ssem1)

    def issue_g(t, p):
        for i in range(KFA):
            pltpu.async_copy(y_sh.at[rowbuf.at[t * KFA + i]], msgbuf.at[p, i], gsem)

    def wait_g(p):
        for i in range(KFA):
            pltpu.make_async_copy(
                y_sh.at[rowbuf.at[i]], msgbuf.at[p, i], gsem
            ).wait()

    def issue_s(t, p):
        for i in range(KFA):
            pltpu.async_copy(
                msgbuf.at[p, i], acc_sh.at[colbuf.at[t * KFA + i]], ssems[p],
                add=True,
            )

    def wait_s(p):
        for i in range(KFA):
            pltpu.make_async_copy(
                msgbuf.at[p, i], acc_sh.at[colbuf.at[i]], ssems[p]
            ).wait()

    issue_g(0, 0)

    def pair(u, _):
        t = 2 * u
        wait_g(0)
        issue_s(t, 0)

        @pl.when(u >= 1)
        def _():
            wait_s(1)
        issue_g(t + 1, 1)
        wait_g(1)
        issue_s(t + 1, 1)

        @pl.when(t + 2 < nb)
        def _():
            wait_s(0)
            issue_g(t + 2, 0)

        return 0

    lax.fori_loop(0, nb // 2, pair, 0)
    wait_s(0)
    wait_s(1)


def _stage_and_init(c, s, seg, y_sh, acc_sh):
    sl = pl.ds(s * RPT, RPT)
    pltpu.sync_copy(seg, y_sh.at[sl, :])

    @pl.when(c == 0)
    def _():
        pltpu.sync_copy(seg, acc_sh.at[sl, :])

    @pl.when(c != 0)
    def _():
        def zb(i, _):
            seg[i, :] = jnp.zeros((F,), jnp.float32)
            return 0

        lax.fori_loop(0, RPT, zb, 0)
        pltpu.sync_copy(seg, acc_sh.at[sl, :])


@functools.partial(
    pl.kernel,
    out_type=(
        jax.ShapeDtypeStruct((NC, NPAD, F), jnp.float32),
        jax.ShapeDtypeStruct((NPAD,), jnp.float32),
        jax.ShapeDtypeStruct((NPAD, F), jnp.float32),
    ),
    mesh=_mesh,
    scratch_types=[
        pltpu.VMEM((KCMX, CHUNK), jnp.int32),
        pltpu.VMEM((KCMX, CHUNK), jnp.int32),
        pltpu.VMEM((2, KFA, CHUNK, F), jnp.float32),
        pltpu.VMEM((RPT, F), jnp.float32),
        pltpu.VMEM((RPT,), jnp.float32),
        pltpu.VMEM((RPT,), jnp.float32),
        pltpu.VMEM((RPT,), jnp.float32),
        pltpu.VMEM((RPT, F), jnp.float32),
        pltpu.VMEM_SHARED((NPAD, F), jnp.float32),
        pltpu.VMEM_SHARED((NPAD, F), jnp.float32),
        pltpu.SemaphoreType.DMA,
        pltpu.SemaphoreType.DMA,
        pltpu.SemaphoreType.DMA,
    ],
    compiler_params=_sc_params,
)
def _agg1_kernel(xw_hbm, degp_hbm, row_hbm, col_hbm, s1p_hbm, dis_hbm,
                 dexp_hbm, rowbuf, colbuf, msgbuf, seg, d0seg, d1seg, disseg,
                 dexp, acc_sh, y_sh, gsem, ssem0, ssem1):
    c = lax.axis_index("c")
    s = lax.axis_index("s")
    wid = s * NC + c
    sl = pl.ds(s * RPT, RPT)
    pltpu.sync_copy(xw_hbm.at[sl, :], seg)
    pltpu.sync_copy(degp_hbm.at[0, sl], d0seg)
    pltpu.sync_copy(degp_hbm.at[1, sl], d1seg)
    @pl.when(c == 0)
    def _():
        pltpu.sync_copy(row_hbm.at[pl.ds(s * KC0, KC0), :],
                        rowbuf.at[pl.ds(0, KC0), :])
        pltpu.sync_copy(col_hbm.at[pl.ds(s * KC0, KC0), :],
                        colbuf.at[pl.ds(0, KC0), :])

    @pl.when(c != 0)
    def _():
        base = NS * KC0 + s * KC1
        pltpu.sync_copy(row_hbm.at[pl.ds(base, KC1), :], rowbuf.at[pl.ds(0, KC1), :])
        pltpu.sync_copy(col_hbm.at[pl.ds(base, KC1), :], colbuf.at[pl.ds(0, KC1), :])

    def dbody(i, _):
        v = d0seg[pl.ds(i * L, L)] + d1seg[pl.ds(i * L, L)] + 1.0
        disseg[pl.ds(i * L, L)] = _vrsqrt(v)
        return 0

    lax.fori_loop(0, RPT // L, dbody, 0)

    def ybody(i, _):
        dv = disseg[pl.ds(i * L, L)]
        for k in range(L):
            r = i * L + k
            seg[r, :] = seg[r, :] * dv[k]
            dexp[r, :] = jax.lax.broadcast_in_dim(dv[k], (F,), ())
        return 0

    lax.fori_loop(0, RPT // L, ybody, 0)

    @pl.when(c == 0)
    def _():
        pltpu.sync_copy(disseg, dis_hbm.at[sl])
        pltpu.sync_copy(dexp, dexp_hbm.at[sl, :])

    _stage_and_init(c, s, seg, y_sh, acc_sh)
    plsc.subcore_barrier()
    nb = jnp.where(c == 0, KC0 // KFA, KC1 // KFA)
    _edge_pipeline(nb, y_sh, acc_sh, rowbuf, colbuf, msgbuf, gsem, ssem0, ssem1)
    plsc.subcore_barrier()
    pltpu.sync_copy(acc_sh.at[sl, :], s1p_hbm.at[c, sl, :])


@functools.partial(
    pl.kernel,
    out_type=jax.ShapeDtypeStruct((NC, NPAD, F), jnp.float32),
    mesh=_mesh,
    scratch_types=[
        pltpu.VMEM((KCMX, CHUNK), jnp.int32),
        pltpu.VMEM((KCMX, CHUNK), jnp.int32),
        pltpu.VMEM((2, KFA, CHUNK, F), jnp.float32),
        pltpu.VMEM((RPT, F), jnp.float32),
        pltpu.VMEM((RPT, F), jnp.float32),
        pltpu.VMEM((RPT,), jnp.float32),
        pltpu.VMEM((F,), jnp.float32),
        pltpu.VMEM_SHARED((NPAD, F), jnp.float32),
        pltpu.VMEM_SHARED((NPAD, F), jnp.float32),
        pltpu.SemaphoreType.DMA,
        pltpu.SemaphoreType.DMA,
        pltpu.SemaphoreType.DMA,
    ],
    compiler_params=_sc_params,
)
def _agg2_kernel(s1p_hbm, dis_hbm, b1_hbm, row_hbm, col_hbm, s2p_hbm,
                 rowbuf, colbuf, msgbuf, seg, p1seg, disseg, b1v,
                 acc_sh, y_sh, gsem, ssem0, ssem1):
    c = lax.axis_index("c")
    s = lax.axis_index("s")
    wid = s * NC + c
    sl = pl.ds(s * RPT, RPT)
    pltpu.sync_copy(s1p_hbm.at[0, sl, :], seg)
    pltpu.sync_copy(s1p_hbm.at[1, sl, :], p1seg)
    pltpu.sync_copy(dis_hbm.at[sl], disseg)
    pltpu.sync_copy(b1_hbm, b1v)
    @pl.when(c == 0)
    def _():
        pltpu.sync_copy(row_hbm.at[pl.ds(s * KC0, KC0), :],
                        rowbuf.at[pl.ds(0, KC0), :])
        pltpu.sync_copy(col_hbm.at[pl.ds(s * KC0, KC0), :],
                        colbuf.at[pl.ds(0, KC0), :])

    @pl.when(c != 0)
    def _():
        base = NS * KC0 + s * KC1
        pltpu.sync_copy(row_hbm.at[pl.ds(base, KC1), :], rowbuf.at[pl.ds(0, KC1), :])
        pltpu.sync_copy(col_hbm.at[pl.ds(base, KC1), :], colbuf.at[pl.ds(0, KC1), :])
    b1r = b1v[...]

    def gbody(i, _):
        dv = disseg[pl.ds(i * L, L)]
        for k in range(L):
            r = i * L + k
            d = dv[k]
            h = jnp.maximum((seg[r, :] + p1seg[r, :]) * d + b1r, 0.0)
            seg[r, :] = h * d
        return 0

    lax.fori_loop(0, RPT // L, gbody, 0)
    _stage_and_init(c, s, seg, y_sh, acc_sh)
    plsc.subcore_barrier()
    nb = jnp.where(c == 0, KC0 // KFA, KC1 // KFA)
    _edge_pipeline(nb, y_sh, acc_sh, rowbuf, colbuf, msgbuf, gsem, ssem0, ssem1)
    plsc.subcore_barrier()
    pltpu.sync_copy(acc_sh.at[sl, :], s2p_hbm.at[c, sl, :])


def _tcmm_body(xp8_ref, w1b_ref, xw8_ref):
    xw8_ref[...] = jnp.dot(
        xp8_ref[...], w1b_ref[...], preferred_element_type=jnp.float32
    )


def _tco_body(s2p8_ref, de8_ref, w2b_ref, b2b_ref, o_ref):
    t8 = (s2p8_ref[0] + s2p8_ref[1]) * de8_ref[...]
    o_ref[...] = (
        jnp.dot(t8, w2b_ref[...], preferred_element_type=jnp.float32)
        + b2b_ref[...]
    )


def _lsm_body(o_ref, out_ref):
    o = o_ref[...]
    m = jnp.max(o, axis=1, keepdims=True)
    e = o - m
    lse = jnp.log(jnp.sum(jnp.exp(e), axis=1, keepdims=True))
    out_ref[...] = e - lse


def kernel(x, edge_index, W1, b1, W2, b2):
    ei = edge_index.astype(jnp.int32)
    e = ei.shape[1]
    pad = EP - e
    fill = jnp.full((pad,), PADI, jnp.int32)
    col2 = jnp.concatenate([ei[1], fill]).reshape(EP // CHUNK, CHUNK)
    row2 = jnp.concatenate([ei[0], fill]).reshape(EP // CHUNK, CHUNK)
    degp = _deg_kernel(col2)
    xp = jnp.pad(x, ((0, NPAD - N), (0, 0)))
    xp8 = jnp.reshape(xp, (NPAD // 8, 8 * 128))
    eye8 = jnp.eye(8, dtype=jnp.float32)
    w1b = (eye8[:, None, :, None] * W1[None, :, None, :]).reshape(8 * 128, 8 * F)
    xw8 = pl.pallas_call(
        _tcmm_body, out_shape=jax.ShapeDtypeStruct((NPAD // 8, 8 * F), jnp.float32)
    )(xp8, w1b)
    xw = jnp.reshape(xw8, (NPAD, F))

    s1p, dis, dexp = _agg1_kernel(xw, degp, row2, col2)
    s2p = _agg2_kernel(s1p, dis, b1, row2, col2)
    s2p8 = jnp.reshape(s2p, (NC, NPAD // 8, 8 * F))
    dexp8 = jnp.reshape(dexp, (NPAD // 8, 8 * F))
    w2b = (eye8[:, None, :, None] * W2[None, :, None, :]).reshape(8 * F, 8 * F2)
    b2b = jnp.tile(b2, (8,)).reshape(1, 8 * F2)
    o512 = pl.pallas_call(
        _tco_body, out_shape=jax.ShapeDtypeStruct((NPAD // 8, 8 * F2), jnp.float32)
    )(s2p8, dexp8, w2b, b2b)
    o = jnp.reshape(o512, (NPAD, F2))
    out = pl.pallas_call(
        _lsm_body, out_shape=jax.ShapeDtypeStruct((NPAD, F2), jnp.float32)
    )(o)
    return out[:N]


# speedup vs baseline: 1.1321x; 1.0560x over previous
import functools

import jax
import jax.numpy as jnp
from jax import lax
from jax.experimental import pallas as pl
from jax.experimental.pallas import tpu as pltpu
from jax.experimental.pallas import tpu_sc as plsc

N = 10000
NPAD = 10240
F = 16
F2 = 64
NC = 2
NS = 16
NW = NC * NS
L = 16
CHUNK = 128
KCH = 80
KF = 8
KFA = 8
KC0 = 96
KC1 = 64
KD0 = 88
KD1 = 72
KCMX = max(KC0, KC1)
KDMX = max(KD0, KD1)
EP = NW * KCH * CHUNK
RPT = NPAD // NS
PADI = N + 16

_mesh = plsc.VectorSubcoreMesh(
    core_axis_name="c", subcore_axis_name="s", num_cores=NC, num_subcores=NS
)
_sc_params = pltpu.CompilerParams(use_tc_tiling_on_sc=False)


def _fill1d(ref, n, val):
    def body(i, _):
        ref[pl.ds(i * L, L)] = jnp.full((L,), val, jnp.float32)
        return 0

    lax.fori_loop(0, n // L, body, 0)


def _vrsqrt(v):
    i = jax.lax.bitcast_convert_type(v, jnp.int32)
    i = jnp.int32(0x5F3759DF) - jax.lax.shift_right_logical(i, 1)
    y = jax.lax.bitcast_convert_type(i, jnp.float32)
    for _ in range(3):
        y = y * (1.5 - 0.5 * v * y * y)
    return y


@functools.partial(
    pl.kernel,
    out_type=jax.ShapeDtypeStruct((NC, NPAD), jnp.float32),
    mesh=_mesh,
    scratch_types=[
        pltpu.VMEM((KDMX, CHUNK), jnp.int32),
        pltpu.VMEM((CHUNK,), jnp.float32),
        pltpu.VMEM((RPT,), jnp.float32),
        pltpu.VMEM_SHARED((NPAD,), jnp.float32),
        pltpu.SemaphoreType.DMA,
    ],
    compiler_params=_sc_params,
)
def _deg_kernel(col_hbm, out_hbm, colbuf, ones_v, zseg, acc_sh, sem):
    c = lax.axis_index("c")
    s = lax.axis_index("s")
    _fill1d(ones_v, CHUNK, 1.0)
    _fill1d(zseg, RPT, 0.0)
    pltpu.sync_copy(zseg, acc_sh.at[pl.ds(s * RPT, RPT)])

    @pl.when(c == 0)
    def _():
        pltpu.sync_copy(col_hbm.at[pl.ds(s * KD0, KD0), :],
                        colbuf.at[pl.ds(0, KD0), :])

    @pl.when(c != 0)
    def _():
        pltpu.sync_copy(
            col_hbm.at[pl.ds(NS * KD0 + s * KD1, KD1), :],
            colbuf.at[pl.ds(0, KD1), :],
        )

    plsc.subcore_barrier()
    nt = jnp.where(c == 0, KD0 // KF, KD1 // KF)

    def body(t, _):
        ds = []
        for i in range(KF):
            j = t * KF + i
            ds.append(pltpu.async_copy(ones_v, acc_sh.at[colbuf.at[j]], sem, add=True))
        for d in ds:
            d.wait()
        return 0

    lax.fori_loop(0, nt, body, 0)
    plsc.subcore_barrier()
    pltpu.sync_copy(acc_sh.at[pl.ds(s * RPT, RPT)], out_hbm.at[c, pl.ds(s * RPT, RPT)])


def _edge_pipeline(nb, y_sh, acc_sh, rowbuf, colbuf, msgbuf, gsem, ssem0, ssem1):
    ssems = (ssem0, ssem1)

    def issue_g(t, p):
        for i in range(KFA):
            pltpu.async_copy(y_sh.at[rowbuf.at[t * KFA + i]], msgbuf.at[p, i], gsem)

    def wait_g(p):
        for i in range(KFA):
            pltpu.make_async_copy(
                y_sh.at[rowbuf.at[i]], msgbuf.at[p, i], gsem
            ).wait()

    def issue_s(t, p):
        for i in range(KFA):
            pltpu.async_copy(
                msgbuf.at[p, i], acc_sh.at[colbuf.at[t * KFA + i]], ssems[p],
                add=True,
            )

    def wait_s(p):
        for i in range(KFA):
            pltpu.make_async_copy(
                msgbuf.at[p, i], acc_sh.at[colbuf.at[i]], ssems[p]
            ).wait()

    issue_g(0, 0)

    def pair(u, _):
        t = 2 * u
        wait_g(0)
        issue_s(t, 0)

        @pl.when(u >= 1)
        def _():
            wait_s(1)
        issue_g(t + 1, 1)
        wait_g(1)
        issue_s(t + 1, 1)

        @pl.when(t + 2 < nb)
        def _():
            wait_s(0)
            issue_g(t + 2, 0)

        return 0

    lax.fori_loop(0, nb // 2, pair, 0)
    wait_s(0)
    wait_s(1)


def _stage_and_init(c, s, seg, y_sh, acc_sh):
    sl = pl.ds(s * RPT, RPT)
    pltpu.sync_copy(seg, y_sh.at[sl, :])

    @pl.when(c == 0)
    def _():
        pltpu.sync_copy(seg, acc_sh.at[sl, :])

    @pl.when(c != 0)
    def _():
        def zb(i, _):
            seg[i, :] = jnp.zeros((F,), jnp.float32)
            return 0

        lax.fori_loop(0, RPT, zb, 0)
        pltpu.sync_copy(seg, acc_sh.at[sl, :])


@functools.partial(
    pl.kernel,
    out_type=(
        jax.ShapeDtypeStruct((NC, NPAD, F), jnp.float32),
        jax.ShapeDtypeStruct((NPAD,), jnp.float32),
        jax.ShapeDtypeStruct((NPAD, F), jnp.float32),
    ),
    mesh=_mesh,
    scratch_types=[
        pltpu.VMEM((KCMX, CHUNK), jnp.int32),
        pltpu.VMEM((KCMX, CHUNK), jnp.int32),
        pltpu.VMEM((2, KFA, CHUNK, F), jnp.float32),
        pltpu.VMEM((RPT, F), jnp.float32),
        pltpu.VMEM((RPT,), jnp.float32),
        pltpu.VMEM((RPT,), jnp.float32),
        pltpu.VMEM((RPT,), jnp.float32),
        pltpu.VMEM((RPT, F), jnp.float32),
        pltpu.VMEM_SHARED((NPAD, F), jnp.float32),
        pltpu.VMEM_SHARED((NPAD, F), jnp.float32),
        pltpu.SemaphoreType.DMA,
        pltpu.SemaphoreType.DMA,
        pltpu.SemaphoreType.DMA,
    ],
    compiler_params=_sc_params,
)
def _agg1_kernel(xw_hbm, degp_hbm, row_hbm, col_hbm, s1p_hbm, dis_hbm,
                 dexp_hbm, rowbuf, colbuf, msgbuf, seg, d0seg, d1seg, disseg,
                 dexp, acc_sh, y_sh, gsem, ssem0, ssem1):
    c = lax.axis_index("c")
    s = lax.axis_index("s")
    wid = s * NC + c
    sl = pl.ds(s * RPT, RPT)
    pltpu.sync_copy(xw_hbm.at[sl, :], seg)
    pltpu.sync_copy(degp_hbm.at[0, sl], d0seg)
    pltpu.sync_copy(degp_hbm.at[1, sl], d1seg)
    @pl.when(c == 0)
    def _():
        pltpu.sync_copy(row_hbm.at[pl.ds(s * KC0, KC0), :],
                        rowbuf.at[pl.ds(0, KC0), :])
        pltpu.sync_copy(col_hbm.at[pl.ds(s * KC0, KC0), :],
                        colbuf.at[pl.ds(0, KC0), :])

    @pl.when(c != 0)
    def _():
        base = NS * KC0 + s * KC1
        pltpu.sync_copy(row_hbm.at[pl.ds(base, KC1), :], rowbuf.at[pl.ds(0, KC1), :])
        pltpu.sync_copy(col_hbm.at[pl.ds(base, KC1), :], colbuf.at[pl.ds(0, KC1), :])

    def dbody(i, _):
        v = d0seg[pl.ds(i * L, L)] + d1seg[pl.ds(i * L, L)] + 1.0
        disseg[pl.ds(i * L, L)] = _vrsqrt(v)
        return 0

    lax.fori_loop(0, RPT // L, dbody, 0)

    def ybody(i, _):
        dv = disseg[pl.ds(i * L, L)]
        for k in range(L):
            r = i * L + k
            seg[r, :] = seg[r, :] * dv[k]
            dexp[r, :] = jax.lax.broadcast_in_dim(dv[k], (F,), ())
        return 0

    lax.fori_loop(0, RPT // L, ybody, 0)

    @pl.when(c == 0)
    def _():
        pltpu.sync_copy(disseg, dis_hbm.at[sl])
        pltpu.sync_copy(dexp, dexp_hbm.at[sl, :])

    _stage_and_init(c, s, seg, y_sh, acc_sh)
    plsc.subcore_barrier()
    nb = jnp.where(c == 0, KC0 // KFA, KC1 // KFA)
    _edge_pipeline(nb, y_sh, acc_sh, rowbuf, colbuf, msgbuf, gsem, ssem0, ssem1)
    plsc.subcore_barrier()
    pltpu.sync_copy(acc_sh.at[sl, :], s1p_hbm.at[c, sl, :])


@functools.partial(
    pl.kernel,
    out_type=jax.ShapeDtypeStruct((NC, NPAD, F), jnp.float32),
    mesh=_mesh,
    scratch_types=[
        pltpu.VMEM((KCMX, CHUNK), jnp.int32),
        pltpu.VMEM((KCMX, CHUNK), jnp.int32),
        pltpu.VMEM((2, KFA, CHUNK, F), jnp.float32),
        pltpu.VMEM((RPT, F), jnp.float32),
        pltpu.VMEM((RPT, F), jnp.float32),
        pltpu.VMEM((RPT,), jnp.float32),
        pltpu.VMEM((F,), jnp.float32),
        pltpu.VMEM_SHARED((NPAD, F), jnp.float32),
        pltpu.VMEM_SHARED((NPAD, F), jnp.float32),
        pltpu.SemaphoreType.DMA,
        pltpu.SemaphoreType.DMA,
        pltpu.SemaphoreType.DMA,
    ],
    compiler_params=_sc_params,
)
def _agg2_kernel(s1p_hbm, dis_hbm, b1_hbm, row_hbm, col_hbm, s2p_hbm,
                 rowbuf, colbuf, msgbuf, seg, p1seg, disseg, b1v,
                 acc_sh, y_sh, gsem, ssem0, ssem1):
    c = lax.axis_index("c")
    s = lax.axis_index("s")
    wid = s * NC + c
    sl = pl.ds(s * RPT, RPT)
    pltpu.sync_copy(s1p_hbm.at[0, sl, :], seg)
    pltpu.sync_copy(s1p_hbm.at[1, sl, :], p1seg)
    pltpu.sync_copy(dis_hbm.at[sl], disseg)
    pltpu.sync_copy(b1_hbm, b1v)
    @pl.when(c == 0)
    def _():
        pltpu.sync_copy(row_hbm.at[pl.ds(s * KC0, KC0), :],
                        rowbuf.at[pl.ds(0, KC0), :])
        pltpu.sync_copy(col_hbm.at[pl.ds(s * KC0, KC0), :],
                        colbuf.at[pl.ds(0, KC0), :])

    @pl.when(c != 0)
    def _():
        base = NS * KC0 + s * KC1
        pltpu.sync_copy(row_hbm.at[pl.ds(base, KC1), :], rowbuf.at[pl.ds(0, KC1), :])
        pltpu.sync_copy(col_hbm.at[pl.ds(base, KC1), :], colbuf.at[pl.ds(0, KC1), :])
    b1r = b1v[...]

    def gbody(i, _):
        dv = disseg[pl.ds(i * L, L)]
        for k in range(L):
            r = i * L + k
            d = dv[k]
            h = jnp.maximum((seg[r, :] + p1seg[r, :]) * d + b1r, 0.0)
            seg[r, :] = h * d
        return 0

    lax.fori_loop(0, RPT // L, gbody, 0)
    _stage_and_init(c, s, seg, y_sh, acc_sh)
    plsc.subcore_barrier()
    nb = jnp.where(c == 0, KC0 // KFA, KC1 // KFA)
    _edge_pipeline(nb, y_sh, acc_sh, rowbuf, colbuf, msgbuf, gsem, ssem0, ssem1)
    plsc.subcore_barrier()
    pltpu.sync_copy(acc_sh.at[sl, :], s2p_hbm.at[c, sl, :])


def _tcmm_body(xp8_ref, w1b_ref, xw8_ref):
    xw8_ref[...] = jnp.dot(
        xp8_ref[...], w1b_ref[...], preferred_element_type=jnp.float32
    )


def _tco_body(s2p8_ref, de8_ref, w2b_ref, b2b_ref, out_ref):
    t8 = (s2p8_ref[0] + s2p8_ref[1]) * de8_ref[...]
    o = (
        jnp.dot(t8, w2b_ref[...], preferred_element_type=jnp.float32)
        + b2b_ref[...]
    )
    parts = []
    for k in range(8):
        sk = jax.lax.slice_in_dim(o, k * F2, (k + 1) * F2, axis=1)
        m = jnp.max(sk, axis=1, keepdims=True)
        e = sk - m
        lse = jnp.log(jnp.sum(jnp.exp(e), axis=1, keepdims=True))
        parts.append(e - lse)
    out_ref[...] = jnp.concatenate(parts, axis=1)


def kernel(x, edge_index, W1, b1, W2, b2):
    ei = edge_index.astype(jnp.int32)
    e = ei.shape[1]
    pad = EP - e
    fill = jnp.full((pad,), PADI, jnp.int32)
    col2 = jnp.concatenate([ei[1], fill]).reshape(EP // CHUNK, CHUNK)
    row2 = jnp.concatenate([ei[0], fill]).reshape(EP // CHUNK, CHUNK)
    degp = _deg_kernel(col2)
    xp = jnp.pad(x, ((0, NPAD - N), (0, 0)))
    xp8 = jnp.reshape(xp, (NPAD // 8, 8 * 128))
    eye8 = jnp.eye(8, dtype=jnp.float32)
    w1b = (eye8[:, None, :, None] * W1[None, :, None, :]).reshape(8 * 128, 8 * F)
    xw8 = pl.pallas_call(
        _tcmm_body, out_shape=jax.ShapeDtypeStruct((NPAD // 8, 8 * F), jnp.float32)
    )(xp8, w1b)
    xw = jnp.reshape(xw8, (NPAD, F))

    s1p, dis, dexp = _agg1_kernel(xw, degp, row2, col2)
    s2p = _agg2_kernel(s1p, dis, b1, row2, col2)
    s2p8 = jnp.reshape(s2p, (NC, NPAD // 8, 8 * F))
    dexp8 = jnp.reshape(dexp, (NPAD // 8, 8 * F))
    w2b = (eye8[:, None, :, None] * W2[None, :, None, :]).reshape(8 * F, 8 * F2)
    b2b = jnp.tile(b2, (8,)).reshape(1, 8 * F2)
    o512 = pl.pallas_call(
        _tco_body, out_shape=jax.ShapeDtypeStruct((NPAD // 8, 8 * F2), jnp.float32)
    )(s2p8, dexp8, w2b, b2b)
    out = jnp.reshape(o512, (NPAD, F2))
    return out[:N]
